# baseline (speedup 1.0000x reference)
import functools

import jax
import jax.numpy as jnp
from jax import lax
from jax.experimental import pallas as pl
from jax.experimental.pallas import tpu as pltpu
from jax.experimental.pallas import tpu_sc as plsc

NF = 16
LANE = 128


@functools.lru_cache(maxsize=None)
def _build(B, N, NC, NS):
    NW = NC * NS
    b_per_w = B // NW
    n_groups = b_per_w // 16
    n_tc = N // LANE
    tail0 = n_tc * LANE
    n_tail = N - tail0
    mesh = plsc.VectorSubcoreMesh(core_axis_name="c", subcore_axis_name="s")

    @functools.partial(
        pl.kernel,
        mesh=mesh,
        out_type=jax.ShapeDtypeStruct((B,), jnp.float32),
        compiler_params=pltpu.CompilerParams(needs_layout_passes=False),
        scratch_types=[
            pltpu.VMEM((1, b_per_w), jnp.int32),
            pltpu.VMEM((1, b_per_w), jnp.int32),
            pltpu.VMEM((16, NF, LANE), jnp.float32),
            pltpu.VMEM((16, NF, LANE), jnp.float32),
            pltpu.VMEM((NF, n_tail), jnp.float32),
            pltpu.VMEM((NF, n_tail), jnp.float32),
            pltpu.VMEM((32,), jnp.float32),
            pltpu.VMEM((b_per_w,), jnp.float32),
            pltpu.SemaphoreType.DMA,
            pltpu.SemaphoreType.DMA,
        ],
    )
    def k(ut_hbm, it_hbm, u2_hbm, i2_hbm, ut_tail_hbm, it_tail_hbm, wb_hbm,
          dummy_hbm, out_hbm, uid_v, iid_v, uslab_v, islab_v, utail_v,
          itail_v, wb_v, out_v, sem_a, sem_b):
        wid = lax.axis_index("s") * NC + lax.axis_index("c")
        base = wid * b_per_w
        pltpu.sync_copy(u2_hbm.at[wid], uid_v)
        pltpu.sync_copy(i2_hbm.at[wid], iid_v)
        pltpu.sync_copy(ut_tail_hbm, utail_v)
        pltpu.sync_copy(it_tail_hbm, itail_v)
        pltpu.sync_copy(wb_hbm, wb_v)

        wv = wb_v[pl.ds(0, 16)]
        bv = wb_v[pl.ds(16, 16)]
        w_s = [wv[f] for f in range(NF)]
        bias = bv[0]
        lane16 = jnp.arange(16, dtype=jnp.int32)

        def extract(uu, vv):
            uoff = uu % LANE
            ioff = vv % LANE
            umask = uu >= tail0
            imask = vv >= tail0
            uto = jnp.clip(uu - tail0, 0, n_tail - 1)
            ito = jnp.clip(vv - tail0, 0, n_tail - 1)
            acc = jnp.full((16,), 0.0, jnp.float32)
            for f in range(NF):
                fv = jnp.full((16,), f, jnp.int32)
                u_f = plsc.load_gather(uslab_v, [lane16, fv, uoff])
                u_f = jnp.where(umask, plsc.load_gather(utail_v, [fv, uto]), u_f)
                i_f = plsc.load_gather(islab_v, [lane16, fv, ioff])
                i_f = jnp.where(imask, plsc.load_gather(itail_v, [fv, ito]), i_f)
                acc = acc + u_f * i_f * w_s[f]
            return acc

        uslab2d = uslab_v.reshape(16 * NF, LANE)
        islab2d = islab_v.reshape(16 * NF, LANE)
        max_base = (n_tc - 1) * LANE

        def g_body(g, carry):
            o = g * 16
            uu = uid_v[0, pl.ds(o, 16)]
            vv = iid_v[0, pl.ds(o, 16)]
            ubase = jnp.minimum(uu & -LANE, max_base)
            ibase = jnp.minimum(vv & -LANE, max_base)
            for j in range(16):
                pltpu.async_copy(
                    ut_hbm.at[:, pl.ds(pl.multiple_of(ubase[j], LANE), LANE)],
                    uslab_v.at[j], sem_a)
                pltpu.async_copy(
                    it_hbm.at[:, pl.ds(pl.multiple_of(ibase[j], LANE), LANE)],
                    islab_v.at[j], sem_a)
            pltpu.make_async_copy(dummy_hbm, uslab2d, sem_a).wait()
            pltpu.make_async_copy(dummy_hbm, islab2d, sem_a).wait()
            out_v[pl.ds(o, 16)] = extract(uu, vv) + bias
            return carry

        lax.fori_loop(0, n_groups, g_body, 0)
        pltpu.sync_copy(out_v, out_hbm.at[pl.ds(base, b_per_w)])

    return k


def kernel(U_ids, I_ids, user_table, item_table, W, b):
    B = U_ids.shape[0]
    N = user_table.shape[0]
    info = plsc.get_sparse_core_info()
    NC, NS = info.num_cores, info.num_subcores
    NW = NC * NS
    u2 = U_ids.astype(jnp.int32).reshape(NW, 1, B // NW)
    i2 = I_ids.astype(jnp.int32).reshape(NW, 1, B // NW)
    wb = jnp.zeros((32,), jnp.float32).at[:NF].set(W.reshape(-1)).at[NF].set(b[0])
    tail0 = (N // LANE) * LANE
    ut_tail = user_table[tail0:, :].T
    it_tail = item_table[tail0:, :].T
    dummy = jnp.zeros((16 * NF, LANE), jnp.float32)
    return _build(B, N, NC, NS)(
        user_table.T, item_table.T, u2, i2, ut_tail, it_tail, wb, dummy)


# speedup vs baseline: 1.0003x; 1.0003x over previous
import functools

import jax
import jax.numpy as jnp
from jax import lax
from jax.experimental import pallas as pl
from jax.experimental.pallas import tpu as pltpu
from jax.experimental.pallas import tpu_sc as plsc

NF = 16
LANE = 128


@functools.lru_cache(maxsize=None)
def _build(B, N, NC, NS):
    NW = NC * NS
    b_per_w = B // NW
    n_groups = b_per_w // 16
    n_tc = N // LANE
    tail0 = n_tc * LANE
    n_tail = N - tail0
    mesh = plsc.VectorSubcoreMesh(core_axis_name="c", subcore_axis_name="s")

    @functools.partial(
        pl.kernel,
        mesh=mesh,
        out_type=jax.ShapeDtypeStruct((B,), jnp.float32),
        compiler_params=pltpu.CompilerParams(needs_layout_passes=False),
        scratch_types=[
            pltpu.VMEM((1, b_per_w), jnp.int32),
            pltpu.VMEM((1, b_per_w), jnp.int32),
            pltpu.VMEM((16, NF, LANE), jnp.float32),
            pltpu.VMEM((16, NF, LANE), jnp.float32),
            pltpu.VMEM((NF, n_tail), jnp.float32),
            pltpu.VMEM((NF, n_tail), jnp.float32),
            pltpu.VMEM((32,), jnp.float32),
            pltpu.VMEM((b_per_w,), jnp.float32),
            pltpu.SemaphoreType.DMA,
        ],
    )
    def k(ut_hbm, it_hbm, u2_hbm, i2_hbm, ut_tail_hbm, it_tail_hbm, wb_hbm,
          dummy_hbm, out_hbm, uid_v, iid_v, uslab_v, islab_v, utail_v,
          itail_v, wb_v, out_v, sem_a):
        wid = lax.axis_index("s") * NC + lax.axis_index("c")
        base = wid * b_per_w
        pltpu.sync_copy(u2_hbm.at[wid], uid_v)
        pltpu.sync_copy(i2_hbm.at[wid], iid_v)
        pltpu.sync_copy(ut_tail_hbm, utail_v)
        pltpu.sync_copy(it_tail_hbm, itail_v)
        pltpu.sync_copy(wb_hbm, wb_v)

        wv = wb_v[pl.ds(0, 16)]
        bv = wb_v[pl.ds(16, 16)]
        w_s = [wv[f] for f in range(NF)]
        bias = bv[0]
        lane16 = jnp.arange(16, dtype=jnp.int32)

        def extract(uu, vv):
            uoff = uu % LANE
            ioff = vv % LANE
            umask = uu >= tail0
            imask = vv >= tail0
            uto = jnp.clip(uu - tail0, 0, n_tail - 1)
            ito = jnp.clip(vv - tail0, 0, n_tail - 1)
            acc = jnp.full((16,), 0.0, jnp.float32)
            for f in range(NF):
                fv = jnp.full((16,), f, jnp.int32)
                u_f = plsc.load_gather(uslab_v, [lane16, fv, uoff])
                u_f = jnp.where(umask, plsc.load_gather(utail_v, [fv, uto]), u_f)
                i_f = plsc.load_gather(islab_v, [lane16, fv, ioff])
                i_f = jnp.where(imask, plsc.load_gather(itail_v, [fv, ito]), i_f)
                acc = acc + u_f * i_f * w_s[f]
            return acc

        uslab2d = uslab_v.reshape(16 * NF, LANE)
        islab2d = islab_v.reshape(16 * NF, LANE)
        max_base = (n_tc - 1) * LANE

        def g_body(g, carry):
            o = g * 16
            uu = uid_v[0, pl.ds(o, 16)]
            vv = iid_v[0, pl.ds(o, 16)]
            ubase = jnp.minimum(uu & -LANE, max_base)
            ibase = jnp.minimum(vv & -LANE, max_base)
            for j in range(16):
                pltpu.async_copy(
                    ut_hbm.at[:, pl.ds(pl.multiple_of(ubase[j], LANE), LANE)],
                    uslab_v.at[j], sem_a)
                pltpu.async_copy(
                    it_hbm.at[:, pl.ds(pl.multiple_of(ibase[j], LANE), LANE)],
                    islab_v.at[j], sem_a)
            pltpu.make_async_copy(dummy_hbm, uslab2d, sem_a).wait()
            pltpu.make_async_copy(dummy_hbm, islab2d, sem_a).wait()
            out_v[pl.ds(o, 16)] = extract(uu, vv) + bias
            return carry

        lax.fori_loop(0, n_groups, g_body, 0)
        pltpu.sync_copy(out_v, out_hbm.at[pl.ds(base, b_per_w)])

    return k


def kernel(U_ids, I_ids, user_table, item_table, W, b):
    B = U_ids.shape[0]
    N = user_table.shape[0]
    info = plsc.get_sparse_core_info()
    NC, NS = info.num_cores, info.num_subcores
    NW = NC * NS
    u2 = U_ids.astype(jnp.int32).reshape(NW, 1, B // NW)
    i2 = I_ids.astype(jnp.int32).reshape(NW, 1, B // NW)
    wb = jnp.zeros((32,), jnp.float32).at[:NF].set(W.reshape(-1)).at[NF].set(b[0])
    tail0 = (N // LANE) * LANE
    ut_tail = user_table[tail0:, :].T
    it_tail = item_table[tail0:, :].T
    dummy = jnp.zeros((16 * NF, LANE), jnp.float32)
    return _build(B, N, NC, NS)(
        user_table.T, item_table.T, u2, i2, ut_tail, it_tail, wb, dummy)
